# baseline (speedup 1.0000x reference)
import functools

import numpy as np
import jax
import jax.numpy as jnp
from jax import lax
from jax.experimental import pallas as pl
from jax.experimental.pallas import tpu as pltpu
from jax.experimental.pallas import tpu_sc as plsc

NUM_FIELDS = 26
FIELD_DIM = 4096
TOTAL_ROWS = NUM_FIELDS * FIELD_DIM
EMBED_DIM = 16
NUM_PAIRS = NUM_FIELDS * (NUM_FIELDS - 1) // 2
NUM_BLOCKS = NUM_PAIRS * 2

NC = 2
NS = 16
NW = NC * NS

RT = FIELD_DIM // 128
KMAX = -(-NUM_BLOCKS // NW)

def _pair_decode(p):
    def step(_, carry):
        i_cur, off = carry
        nxt = off + (NUM_FIELDS - 1 - i_cur)
        take = nxt <= p
        return (jnp.where(take, i_cur + 1, i_cur), jnp.where(take, nxt, off))

    i, off = lax.fori_loop(0, NUM_FIELDS - 1, step, (jnp.int32(0), jnp.int32(0)))
    j = i + 1 + (p - off)
    return i, j


def _rows(g):
    dt = lax.rem(g, 2)
    p = lax.div(g, 2)
    i, j = _pair_decode(p)
    row_a = (j * 2 + dt) * (RT * NUM_FIELDS) + i * RT
    row_b = (i * 2 + dt) * (RT * NUM_FIELDS) + j * RT
    return row_a, row_b, i, j


def _body(tv, xt, out, xa_v, xb_v, a_v, b_v, ov, sem_a, sem_b, sem_out):
    wid = lax.axis_index("s") * NC + lax.axis_index("c")

    def fire_a(g):
        row_a, _, i, _ = _rows(g)
        pltpu.async_copy(tv.at[pl.ds(row_a * 1024, RT * 1024)], a_v, sem_a)
        pltpu.async_copy(xt.at[i], xa_v, sem_a)

    def fire_b(g):
        _, row_b, _, j = _rows(g)
        pltpu.async_copy(tv.at[pl.ds(row_b * 1024, RT * 1024)], b_v, sem_b)
        pltpu.async_copy(xt.at[j], xb_v, sem_b)

    fire_a(wid)
    fire_b(wid)

    @pl.loop(0, KMAX)
    def _k(k):
        g = k * NW + wid

        @pl.when(g < NUM_BLOCKS)
        def _block():
            pltpu.make_async_copy(tv.at[pl.ds(0, RT * 1024)], a_v, sem_a).wait()
            pltpu.make_async_copy(xt.at[0], xa_v, sem_a).wait()

            @pl.when(k > 0)
            def _():
                pltpu.make_async_copy(ov, out.at[0], sem_out).wait()

            @plsc.parallel_loop(0, RT, unroll=4)
            def _bta(bt):
                for s in range(8):
                    xi = xa_v[pl.ds(bt * 128 + s * 16, 16)]
                    fa = jax.lax.shift_left(
                        jax.lax.shift_right_logical(xi, 7), 10) \
                        + jax.lax.bitwise_and(xi, 127)
                    for dl in range(8):
                        va = plsc.load_gather(a_v, [fa + dl * 128])
                        ov[pl.ds(bt * 1024 + dl * 128 + s * 16, 16)] = va

            @pl.when(g + NW < NUM_BLOCKS)
            def _():
                fire_a(g + NW)

            pltpu.make_async_copy(tv.at[pl.ds(0, RT * 1024)], b_v, sem_b).wait()
            pltpu.make_async_copy(xt.at[0], xb_v, sem_b).wait()

            @plsc.parallel_loop(0, RT, unroll=4)
            def _btb(bt):
                for s in range(8):
                    xj = xb_v[pl.ds(bt * 128 + s * 16, 16)]
                    fb = jax.lax.shift_left(
                        jax.lax.shift_right_logical(xj, 7), 10) \
                        + jax.lax.bitwise_and(xj, 127)
                    for dl in range(8):
                        vb = plsc.load_gather(b_v, [fb + dl * 128])
                        pos = pl.ds(bt * 1024 + dl * 128 + s * 16, 16)
                        ov[pos] = ov[pos] * vb

            @pl.when(g + NW < NUM_BLOCKS)
            def _():
                fire_b(g + NW)

            pltpu.async_copy(ov, out.at[g], sem_out)

    pltpu.make_async_copy(ov, out.at[0], sem_out).wait()


def kernel(x, tables):
    batch = x.shape[0]
    xt = x.astype(jnp.int32).T
    tv = (tables.reshape(NUM_FIELDS, TOTAL_ROWS // 128, 128, 2, 8)
          .transpose(0, 3, 1, 4, 2)
          .reshape(NUM_FIELDS * 2 * (TOTAL_ROWS // 128) * 1024))
    fn = pl.kernel(
        _body,
        out_type=jax.ShapeDtypeStruct((NUM_BLOCKS, batch // 128 * 1024),
                                      jnp.float32),
        mesh=plsc.VectorSubcoreMesh(core_axis_name="c", subcore_axis_name="s"),
        scratch_types=[
            pltpu.VMEM((batch,), jnp.int32),
            pltpu.VMEM((batch,), jnp.int32),
            pltpu.VMEM((RT * 1024,), jnp.float32),
            pltpu.VMEM((RT * 1024,), jnp.float32),
            pltpu.VMEM((batch // 128 * 1024,), jnp.float32),
            pltpu.SemaphoreType.DMA,
            pltpu.SemaphoreType.DMA,
            pltpu.SemaphoreType.DMA,
        ],
        compiler_params=pltpu.CompilerParams(use_tc_tiling_on_sc=False,
                                             needs_layout_passes=False),
    )
    o4 = fn(tv, xt)
    out = (o4.reshape(NUM_PAIRS, 2, batch // 128, 8, 128)
           .transpose(2, 4, 0, 1, 3)
           .reshape(batch, NUM_PAIRS, EMBED_DIM))
    return out


# speedup vs baseline: 1.6308x; 1.6308x over previous
import functools

import numpy as np
import jax
import jax.numpy as jnp
from jax import lax
from jax.experimental import pallas as pl
from jax.experimental.pallas import tpu as pltpu
from jax.experimental.pallas import tpu_sc as plsc

NUM_FIELDS = 26
FIELD_DIM = 4096
TOTAL_ROWS = NUM_FIELDS * FIELD_DIM
EMBED_DIM = 16
NUM_PAIRS = NUM_FIELDS * (NUM_FIELDS - 1) // 2
NUM_BLOCKS = NUM_PAIRS * 2

NC = 2
NS = 16
NW = NC * NS

RT = FIELD_DIM // 128
KMAX = -(-NUM_BLOCKS // NW)

def _pair_decode(p):
    def step(_, carry):
        i_cur, off = carry
        nxt = off + (NUM_FIELDS - 1 - i_cur)
        take = nxt <= p
        return (jnp.where(take, i_cur + 1, i_cur), jnp.where(take, nxt, off))

    i, off = lax.fori_loop(0, NUM_FIELDS - 1, step, (jnp.int32(0), jnp.int32(0)))
    j = i + 1 + (p - off)
    return i, j


def _rows(g):
    dt = lax.rem(g, 2)
    p = lax.div(g, 2)
    i, j = _pair_decode(p)
    row_a = (j * 2 + dt) * (RT * NUM_FIELDS) + i * RT
    row_b = (i * 2 + dt) * (RT * NUM_FIELDS) + j * RT
    return row_a, row_b, i, j


def _body(tv, xt, out, xa_v, xb_v, a_v, b_v, ov, sem_a, sem_b, sem_out):
    wid = lax.axis_index("s") * NC + lax.axis_index("c")

    def fire_a(g):
        row_a, _, i, _ = _rows(g)
        pltpu.async_copy(tv.at[pl.ds(row_a * 1024, RT * 1024)], a_v, sem_a)
        pltpu.async_copy(xt.at[i], xa_v, sem_a)

    def fire_b(g):
        _, row_b, _, j = _rows(g)
        pltpu.async_copy(tv.at[pl.ds(row_b * 1024, RT * 1024)], b_v, sem_b)
        pltpu.async_copy(xt.at[j], xb_v, sem_b)

    fire_a(wid)
    fire_b(wid)

    @pl.loop(0, KMAX)
    def _k(k):
        g = k * NW + wid

        @pl.when(g < NUM_BLOCKS)
        def _block():
            pltpu.make_async_copy(tv.at[pl.ds(0, RT * 1024)], a_v, sem_a).wait()
            pltpu.make_async_copy(xt.at[0], xa_v, sem_a).wait()

            @pl.when(k > 0)
            def _():
                pltpu.make_async_copy(ov, out.at[0], sem_out).wait()

            @plsc.parallel_loop(0, RT * 8, unroll=2)
            def _bta(q):
                base = jax.lax.shift_left(jax.lax.shift_right_logical(q, 3),
                                          10) + jax.lax.bitwise_and(q, 7) * 16
                xi = xa_v[pl.ds(q * 16, 16)]
                fa = jax.lax.shift_left(
                    jax.lax.shift_right_logical(xi, 7), 10) \
                    + jax.lax.bitwise_and(xi, 127)
                for dl in range(8):
                    va = plsc.load_gather(a_v, [fa + dl * 128])
                    ov[pl.ds(base + dl * 128, 16)] = va

            @pl.when(g + NW < NUM_BLOCKS)
            def _():
                fire_a(g + NW)

            pltpu.make_async_copy(tv.at[pl.ds(0, RT * 1024)], b_v, sem_b).wait()
            pltpu.make_async_copy(xt.at[0], xb_v, sem_b).wait()

            @plsc.parallel_loop(0, RT * 8, unroll=2)
            def _btb(q):
                base = jax.lax.shift_left(jax.lax.shift_right_logical(q, 3),
                                          10) + jax.lax.bitwise_and(q, 7) * 16
                xj = xb_v[pl.ds(q * 16, 16)]
                fb = jax.lax.shift_left(
                    jax.lax.shift_right_logical(xj, 7), 10) \
                    + jax.lax.bitwise_and(xj, 127)
                for dl in range(8):
                    vb = plsc.load_gather(b_v, [fb + dl * 128])
                    pos = pl.ds(base + dl * 128, 16)
                    ov[pos] = ov[pos] * vb

            @pl.when(g + NW < NUM_BLOCKS)
            def _():
                fire_b(g + NW)

            pltpu.async_copy(ov, out.at[g], sem_out)

    pltpu.make_async_copy(ov, out.at[0], sem_out).wait()


def kernel(x, tables):
    batch = x.shape[0]
    xt = x.astype(jnp.int32).T
    tv = (tables.reshape(NUM_FIELDS, TOTAL_ROWS // 128, 128, 2, 8)
          .transpose(0, 3, 1, 4, 2)
          .reshape(NUM_FIELDS * 2 * (TOTAL_ROWS // 128) * 1024))
    fn = pl.kernel(
        _body,
        out_type=jax.ShapeDtypeStruct((NUM_BLOCKS, batch // 128 * 1024),
                                      jnp.float32),
        mesh=plsc.VectorSubcoreMesh(core_axis_name="c", subcore_axis_name="s"),
        scratch_types=[
            pltpu.VMEM((batch,), jnp.int32),
            pltpu.VMEM((batch,), jnp.int32),
            pltpu.VMEM((RT * 1024,), jnp.float32),
            pltpu.VMEM((RT * 1024,), jnp.float32),
            pltpu.VMEM((batch // 128 * 1024,), jnp.float32),
            pltpu.SemaphoreType.DMA,
            pltpu.SemaphoreType.DMA,
            pltpu.SemaphoreType.DMA,
        ],
        compiler_params=pltpu.CompilerParams(use_tc_tiling_on_sc=False,
                                             needs_layout_passes=False),
    )
    o4 = fn(tv, xt)
    out = (o4.reshape(NUM_PAIRS, 2, batch // 128, 8, 128)
           .transpose(2, 4, 0, 1, 3)
           .reshape(batch, NUM_PAIRS, EMBED_DIM))
    return out


# speedup vs baseline: 1.6381x; 1.0045x over previous
import functools

import numpy as np
import jax
import jax.numpy as jnp
from jax import lax
from jax.experimental import pallas as pl
from jax.experimental.pallas import tpu as pltpu
from jax.experimental.pallas import tpu_sc as plsc

NUM_FIELDS = 26
FIELD_DIM = 4096
TOTAL_ROWS = NUM_FIELDS * FIELD_DIM
EMBED_DIM = 16
NUM_PAIRS = NUM_FIELDS * (NUM_FIELDS - 1) // 2
NUM_BLOCKS = NUM_PAIRS * 2

NC = 2
NS = 16
NW = NC * NS

RT = FIELD_DIM // 128
KMAX = -(-NUM_BLOCKS // NW)

def _pair_decode(p):
    def step(_, carry):
        i_cur, off = carry
        nxt = off + (NUM_FIELDS - 1 - i_cur)
        take = nxt <= p
        return (jnp.where(take, i_cur + 1, i_cur), jnp.where(take, nxt, off))

    i, off = lax.fori_loop(0, NUM_FIELDS - 1, step, (jnp.int32(0), jnp.int32(0)))
    j = i + 1 + (p - off)
    return i, j


def _rows(g):
    dt = lax.rem(g, 2)
    p = lax.div(g, 2)
    i, j = _pair_decode(p)
    row_a = (j * 2 + dt) * (RT * NUM_FIELDS) + i * RT
    row_b = (i * 2 + dt) * (RT * NUM_FIELDS) + j * RT
    return row_a, row_b, i, j


def _body(tv, xt, out, xa_v, xb_v, a_v, b_v, ov, sem_a, sem_b, sem_out):
    wid = lax.axis_index("s") * NC + lax.axis_index("c")

    def fire_a(g):
        row_a, _, i, _ = _rows(g)
        pltpu.async_copy(tv.at[pl.ds(row_a * 1024, RT * 1024)], a_v, sem_a)
        pltpu.async_copy(xt.at[i], xa_v, sem_a)

    def fire_b(g):
        _, row_b, _, j = _rows(g)
        pltpu.async_copy(tv.at[pl.ds(row_b * 1024, RT * 1024)], b_v, sem_b)
        pltpu.async_copy(xt.at[j], xb_v, sem_b)

    fire_a(wid)
    fire_b(wid)

    @pl.loop(0, KMAX)
    def _k(k):
        g = k * NW + wid

        @pl.when(g < NUM_BLOCKS)
        def _block():
            pltpu.make_async_copy(tv.at[pl.ds(0, RT * 1024)], a_v, sem_a).wait()
            pltpu.make_async_copy(xt.at[0], xa_v, sem_a).wait()

            @pl.when(k > 0)
            def _():
                pltpu.make_async_copy(ov, out.at[0], sem_out).wait()

            @plsc.parallel_loop(0, RT * 8, unroll=4)
            def _bta(q):
                base = jax.lax.shift_left(jax.lax.shift_right_logical(q, 3),
                                          10) + jax.lax.bitwise_and(q, 7) * 16
                xi = xa_v[pl.ds(q * 16, 16)]
                fa = jax.lax.shift_left(
                    jax.lax.shift_right_logical(xi, 7), 10) \
                    + jax.lax.bitwise_and(xi, 127)
                for dl in range(8):
                    va = plsc.load_gather(a_v, [fa + dl * 128])
                    ov[pl.ds(base + dl * 128, 16)] = va

            @pl.when(g + NW < NUM_BLOCKS)
            def _():
                fire_a(g + NW)

            pltpu.make_async_copy(tv.at[pl.ds(0, RT * 1024)], b_v, sem_b).wait()
            pltpu.make_async_copy(xt.at[0], xb_v, sem_b).wait()

            @plsc.parallel_loop(0, RT * 8, unroll=4)
            def _btb(q):
                base = jax.lax.shift_left(jax.lax.shift_right_logical(q, 3),
                                          10) + jax.lax.bitwise_and(q, 7) * 16
                xj = xb_v[pl.ds(q * 16, 16)]
                fb = jax.lax.shift_left(
                    jax.lax.shift_right_logical(xj, 7), 10) \
                    + jax.lax.bitwise_and(xj, 127)
                for dl in range(8):
                    vb = plsc.load_gather(b_v, [fb + dl * 128])
                    pos = pl.ds(base + dl * 128, 16)
                    ov[pos] = ov[pos] * vb

            @pl.when(g + NW < NUM_BLOCKS)
            def _():
                fire_b(g + NW)

            pltpu.async_copy(ov, out.at[g], sem_out)

    pltpu.make_async_copy(ov, out.at[0], sem_out).wait()


def kernel(x, tables):
    batch = x.shape[0]
    xt = x.astype(jnp.int32).T
    tv = (tables.reshape(NUM_FIELDS, TOTAL_ROWS // 128, 128, 2, 8)
          .transpose(0, 3, 1, 4, 2)
          .reshape(NUM_FIELDS * 2 * (TOTAL_ROWS // 128) * 1024))
    fn = pl.kernel(
        _body,
        out_type=jax.ShapeDtypeStruct((NUM_BLOCKS, batch // 128 * 1024),
                                      jnp.float32),
        mesh=plsc.VectorSubcoreMesh(core_axis_name="c", subcore_axis_name="s"),
        scratch_types=[
            pltpu.VMEM((batch,), jnp.int32),
            pltpu.VMEM((batch,), jnp.int32),
            pltpu.VMEM((RT * 1024,), jnp.float32),
            pltpu.VMEM((RT * 1024,), jnp.float32),
            pltpu.VMEM((batch // 128 * 1024,), jnp.float32),
            pltpu.SemaphoreType.DMA,
            pltpu.SemaphoreType.DMA,
            pltpu.SemaphoreType.DMA,
        ],
        compiler_params=pltpu.CompilerParams(use_tc_tiling_on_sc=False,
                                             needs_layout_passes=False),
    )
    o4 = fn(tv, xt)
    out = (o4.reshape(NUM_PAIRS, 2, batch // 128, 8, 128)
           .transpose(2, 4, 0, 1, 3)
           .reshape(batch, NUM_PAIRS, EMBED_DIM))
    return out


# speedup vs baseline: 1.6450x; 1.0042x over previous
import functools

import numpy as np
import jax
import jax.numpy as jnp
from jax import lax
from jax.experimental import pallas as pl
from jax.experimental.pallas import tpu as pltpu
from jax.experimental.pallas import tpu_sc as plsc

NUM_FIELDS = 26
FIELD_DIM = 4096
TOTAL_ROWS = NUM_FIELDS * FIELD_DIM
EMBED_DIM = 16
NUM_PAIRS = NUM_FIELDS * (NUM_FIELDS - 1) // 2
NUM_BLOCKS = NUM_PAIRS * 2

NC = 2
NS = 16
NW = NC * NS

RT = FIELD_DIM // 128
KMAX = -(-NUM_BLOCKS // NW)

def _pair_decode(p):
    def step(_, carry):
        i_cur, off = carry
        nxt = off + (NUM_FIELDS - 1 - i_cur)
        take = nxt <= p
        return (jnp.where(take, i_cur + 1, i_cur), jnp.where(take, nxt, off))

    i, off = lax.fori_loop(0, NUM_FIELDS - 1, step, (jnp.int32(0), jnp.int32(0)))
    j = i + 1 + (p - off)
    return i, j


def _rows(g):
    dt = lax.rem(g, 2)
    p = lax.div(g, 2)
    i, j = _pair_decode(p)
    row_a = (j * 2 + dt) * (RT * NUM_FIELDS) + i * RT
    row_b = (i * 2 + dt) * (RT * NUM_FIELDS) + j * RT
    return row_a, row_b, i, j


def _body(tv, xt, out, xa_v, xb_v, a_v, b_v, ov, sem_a, sem_b, sem_out):
    wid = lax.axis_index("s") * NC + lax.axis_index("c")

    def fire_a(g):
        row_a, _, i, _ = _rows(g)
        pltpu.async_copy(tv.at[pl.ds(row_a * 1024, RT * 1024)], a_v, sem_a)
        pltpu.async_copy(xt.at[i], xa_v, sem_a)

    def fire_b(g):
        _, row_b, _, j = _rows(g)
        pltpu.async_copy(tv.at[pl.ds(row_b * 1024, RT * 1024)], b_v, sem_b)
        pltpu.async_copy(xt.at[j], xb_v, sem_b)

    fire_a(wid)
    fire_b(wid)

    @pl.loop(0, KMAX)
    def _k(k):
        g = k * NW + wid

        @pl.when(g < NUM_BLOCKS)
        def _block():
            pltpu.make_async_copy(tv.at[pl.ds(0, RT * 1024)], a_v, sem_a).wait()
            pltpu.make_async_copy(xt.at[0], xa_v, sem_a).wait()

            @pl.when(k > 0)
            def _():
                pltpu.make_async_copy(ov, out.at[0], sem_out).wait()

            @plsc.parallel_loop(0, RT * 8, unroll=8)
            def _bta(q):
                base = jax.lax.shift_left(jax.lax.shift_right_logical(q, 3),
                                          10) + jax.lax.bitwise_and(q, 7) * 16
                xi = xa_v[pl.ds(q * 16, 16)]
                fa = jax.lax.shift_left(
                    jax.lax.shift_right_logical(xi, 7), 10) \
                    + jax.lax.bitwise_and(xi, 127)
                for dl in range(8):
                    va = plsc.load_gather(a_v, [fa + dl * 128])
                    ov[pl.ds(base + dl * 128, 16)] = va

            @pl.when(g + NW < NUM_BLOCKS)
            def _():
                fire_a(g + NW)

            pltpu.make_async_copy(tv.at[pl.ds(0, RT * 1024)], b_v, sem_b).wait()
            pltpu.make_async_copy(xt.at[0], xb_v, sem_b).wait()

            @plsc.parallel_loop(0, RT * 8, unroll=8)
            def _btb(q):
                base = jax.lax.shift_left(jax.lax.shift_right_logical(q, 3),
                                          10) + jax.lax.bitwise_and(q, 7) * 16
                xj = xb_v[pl.ds(q * 16, 16)]
                fb = jax.lax.shift_left(
                    jax.lax.shift_right_logical(xj, 7), 10) \
                    + jax.lax.bitwise_and(xj, 127)
                for dl in range(8):
                    vb = plsc.load_gather(b_v, [fb + dl * 128])
                    pos = pl.ds(base + dl * 128, 16)
                    ov[pos] = ov[pos] * vb

            @pl.when(g + NW < NUM_BLOCKS)
            def _():
                fire_b(g + NW)

            pltpu.async_copy(ov, out.at[g], sem_out)

    pltpu.make_async_copy(ov, out.at[0], sem_out).wait()


def kernel(x, tables):
    batch = x.shape[0]
    xt = x.astype(jnp.int32).T
    tv = (tables.reshape(NUM_FIELDS, TOTAL_ROWS // 128, 128, 2, 8)
          .transpose(0, 3, 1, 4, 2)
          .reshape(NUM_FIELDS * 2 * (TOTAL_ROWS // 128) * 1024))
    fn = pl.kernel(
        _body,
        out_type=jax.ShapeDtypeStruct((NUM_BLOCKS, batch // 128 * 1024),
                                      jnp.float32),
        mesh=plsc.VectorSubcoreMesh(core_axis_name="c", subcore_axis_name="s"),
        scratch_types=[
            pltpu.VMEM((batch,), jnp.int32),
            pltpu.VMEM((batch,), jnp.int32),
            pltpu.VMEM((RT * 1024,), jnp.float32),
            pltpu.VMEM((RT * 1024,), jnp.float32),
            pltpu.VMEM((batch // 128 * 1024,), jnp.float32),
            pltpu.SemaphoreType.DMA,
            pltpu.SemaphoreType.DMA,
            pltpu.SemaphoreType.DMA,
        ],
        compiler_params=pltpu.CompilerParams(use_tc_tiling_on_sc=False,
                                             needs_layout_passes=False),
    )
    o4 = fn(tv, xt)
    out = (o4.reshape(NUM_PAIRS, 2, batch // 128, 8, 128)
           .transpose(2, 4, 0, 1, 3)
           .reshape(batch, NUM_PAIRS, EMBED_DIM))
    return out
